# baseline (speedup 1.0000x reference)
import functools

import jax
import jax.numpy as jnp
from jax import lax
from jax.experimental import pallas as pl
from jax.experimental.pallas import tpu as pltpu
from jax.experimental.pallas import tpu_sc as plsc

_IDX_CHUNK = 128


def _make_sc_gather(V, D, B):
    info = plsc.get_sparse_core_info()
    NC, NS = info.num_cores, info.num_subcores
    NW = NC * NS
    assert B % (NW * _IDX_CHUNK) == 0
    b_per_w = B // NW
    n_chunks = b_per_w // _IDX_CHUNK
    mesh = plsc.VectorSubcoreMesh(core_axis_name="c", subcore_axis_name="s")

    @functools.partial(
        pl.kernel,
        mesh=mesh,
        out_type=jax.ShapeDtypeStruct((B, D), jnp.float32),
        scratch_types=[
            pltpu.VMEM((n_chunks, _IDX_CHUNK), jnp.int32),
            pltpu.VMEM((b_per_w, D), jnp.float32),
            pltpu.SemaphoreType.DMA,
        ],
        compiler_params=pltpu.CompilerParams(use_tc_tiling_on_sc=False),
    )
    def gather(table_hbm, idx_hbm, out_hbm, idx_v, rows_v, sem):
        wid = lax.axis_index("s") * NC + lax.axis_index("c")
        base = wid * b_per_w
        pltpu.sync_copy(idx_hbm.at[pl.ds(wid * n_chunks, n_chunks)], idx_v)
        copies = []
        for j in range(n_chunks):
            copies.append(
                pltpu.make_async_copy(
                    table_hbm.at[idx_v.at[j]],
                    rows_v.at[pl.ds(j * _IDX_CHUNK, _IDX_CHUNK)],
                    sem,
                )
            )
        for c in copies:
            c.start()
        for c in copies:
            c.wait()
        pltpu.sync_copy(rows_v, out_hbm.at[pl.ds(base, b_per_w)])

    return gather


def _mlp_body(emb_ref, gen_ref, w1a_ref, w1b_ref, b1_ref, w2_ref, b2_ref,
              out_ref):
    h = jnp.dot(emb_ref[...], w1a_ref[...], preferred_element_type=jnp.float32)
    h = h + jnp.dot(gen_ref[...], w1b_ref[...],
                    preferred_element_type=jnp.float32)
    h = jnp.maximum(h + b1_ref[...], 0.0)
    y = jnp.dot(h, w2_ref[...], preferred_element_type=jnp.float32)
    y = y + b2_ref[...]
    norm = jnp.sqrt(jnp.sum(y * y, axis=-1, keepdims=True))
    out_ref[...] = y / jnp.maximum(norm, 1e-12)


def kernel(item_ids, genre_vectors, table, W1, b1, W2, b2):
    B, = item_ids.shape
    V, D = table.shape
    G = genre_vectors.shape[1]
    H = W1.shape[1]

    idx2d = item_ids.astype(jnp.int32).reshape(B // _IDX_CHUNK, _IDX_CHUNK)
    emb = _make_sc_gather(V, D, B)(table, idx2d)

    W1a = W1[:D]
    W1b = W1[D:]
    b1r = b1.reshape(1, H)
    b2r = b2.reshape(1, D)

    BB = 2048
    grid = (B // BB,)
    out = pl.pallas_call(
        _mlp_body,
        grid=grid,
        in_specs=[
            pl.BlockSpec((BB, D), lambda i: (i, 0)),
            pl.BlockSpec((BB, G), lambda i: (i, 0)),
            pl.BlockSpec((D, H), lambda i: (0, 0)),
            pl.BlockSpec((G, H), lambda i: (0, 0)),
            pl.BlockSpec((1, H), lambda i: (0, 0)),
            pl.BlockSpec((H, D), lambda i: (0, 0)),
            pl.BlockSpec((1, D), lambda i: (0, 0)),
        ],
        out_specs=pl.BlockSpec((BB, D), lambda i: (i, 0)),
        out_shape=jax.ShapeDtypeStruct((B, D), jnp.float32),
    )(emb, genre_vectors, W1a, W1b, b1r, W2, b2r)
    return out


# speedup vs baseline: 1.7216x; 1.7216x over previous
import functools

import jax
import jax.numpy as jnp
from jax import lax
from jax.experimental import pallas as pl
from jax.experimental.pallas import tpu as pltpu
from jax.experimental.pallas import tpu_sc as plsc

_IDX_CHUNK = 128
_PK = 512000
_BBK = 2048


def _repack_body(t1_ref, t2_ref, out_ref):
    out_ref[:, 0:64] = jnp.transpose(t1_ref[...])
    out_ref[:, 64:128] = jnp.transpose(t2_ref[...])


def _make_repack(V, D):
    n_blocks = _PK // _BBK
    last_blk = pl.cdiv(V, _BBK) - 1

    return pl.pallas_call(
        _repack_body,
        grid=(n_blocks,),
        in_specs=[
            pl.BlockSpec((D, _BBK), lambda i: (0, i)),
            pl.BlockSpec((D, _BBK),
                         lambda i: (0, jnp.minimum(i + _PK // _BBK,
                                                   last_blk))),
        ],
        out_specs=pl.BlockSpec((_BBK, 2 * D), lambda i: (i, 0)),
        out_shape=jax.ShapeDtypeStruct((_PK, 2 * D), jnp.float32),
    )


def _make_sc_gather(D2, B):
    info = plsc.get_sparse_core_info()
    NC, NS = info.num_cores, info.num_subcores
    NW = NC * NS
    assert B % (NW * _IDX_CHUNK) == 0
    b_per_w = B // NW
    n_chunks = b_per_w // _IDX_CHUNK
    mesh = plsc.VectorSubcoreMesh(core_axis_name="c", subcore_axis_name="s")

    @functools.partial(
        pl.kernel,
        mesh=mesh,
        out_type=jax.ShapeDtypeStruct((B, D2), jnp.float32),
        scratch_types=[
            pltpu.VMEM((n_chunks, _IDX_CHUNK), jnp.int32),
            pltpu.VMEM((b_per_w, D2), jnp.float32),
            pltpu.SemaphoreType.DMA,
        ],
        compiler_params=pltpu.CompilerParams(use_tc_tiling_on_sc=True),
    )
    def gather(p_hbm, idx_hbm, out_hbm, idx_v, rows_v, sem):
        wid = lax.axis_index("s") * NC + lax.axis_index("c")
        base = wid * b_per_w
        pltpu.sync_copy(idx_hbm.at[pl.ds(wid * n_chunks, n_chunks)], idx_v)
        copies = []
        for j in range(n_chunks):
            copies.append(
                pltpu.make_async_copy(
                    p_hbm.at[idx_v.at[j]],
                    rows_v.at[pl.ds(j * _IDX_CHUNK, _IDX_CHUNK)],
                    sem,
                )
            )
        for cp in copies:
            cp.start()
        for cp in copies:
            cp.wait()
        pltpu.sync_copy(rows_v, out_hbm.at[pl.ds(base, b_per_w)])

    return gather


def _mlp_body(x_ref, hi_ref, gen_ref, w1a_ref, w1b_ref, b1_ref, w2_ref,
              b2_ref, out_ref):
    hi = jnp.transpose(hi_ref[...]) > 0
    emb = jnp.where(hi, x_ref[:, 64:128], x_ref[:, 0:64])
    h = jnp.dot(emb, w1a_ref[...], preferred_element_type=jnp.float32)
    h = h + jnp.dot(gen_ref[...], w1b_ref[...],
                    preferred_element_type=jnp.float32)
    h = jnp.maximum(h + b1_ref[...], 0.0)
    y = jnp.dot(h, w2_ref[...], preferred_element_type=jnp.float32)
    y = y + b2_ref[...]
    norm = jnp.sqrt(jnp.sum(y * y, axis=-1, keepdims=True))
    out_ref[...] = y / jnp.maximum(norm, 1e-12)


def kernel(item_ids, genre_vectors, table, W1, b1, W2, b2):
    B, = item_ids.shape
    V, D = table.shape
    G = genre_vectors.shape[1]
    H = W1.shape[1]

    packed = _make_repack(V, D)(table.T, table.T)

    ids = item_ids.astype(jnp.int32)
    p_idx = jnp.where(ids < _PK, ids, ids - _PK)
    idx2d = p_idx.reshape(B // _IDX_CHUNK, _IDX_CHUNK)
    x128 = _make_sc_gather(2 * D, B)(packed, idx2d)

    W1a = W1[:D]
    W1b = W1[D:]
    b1r = b1.reshape(1, H)
    b2r = b2.reshape(1, D)
    hi2d = (ids >= _PK).astype(jnp.int32).reshape(1, B)

    BB = 2048
    grid = (B // BB,)
    out = pl.pallas_call(
        _mlp_body,
        grid=grid,
        in_specs=[
            pl.BlockSpec((BB, 2 * D), lambda i: (i, 0)),
            pl.BlockSpec((1, BB), lambda i: (0, i)),
            pl.BlockSpec((BB, G), lambda i: (i, 0)),
            pl.BlockSpec((D, H), lambda i: (0, 0)),
            pl.BlockSpec((G, H), lambda i: (0, 0)),
            pl.BlockSpec((1, H), lambda i: (0, 0)),
            pl.BlockSpec((H, D), lambda i: (0, 0)),
            pl.BlockSpec((1, D), lambda i: (0, 0)),
        ],
        out_specs=pl.BlockSpec((BB, D), lambda i: (i, 0)),
        out_shape=jax.ShapeDtypeStruct((B, D), jnp.float32),
    )(x128, hi2d, genre_vectors, W1a, W1b, b1r, W2, b2r)
    return out


# speedup vs baseline: 1.7218x; 1.0001x over previous
import functools

import jax
import jax.numpy as jnp
from jax import lax
from jax.experimental import pallas as pl
from jax.experimental.pallas import tpu as pltpu
from jax.experimental.pallas import tpu_sc as plsc

_IDX_CHUNK = 128
_PK = 512000
_BBK = 2048


def _repack_body(t1_ref, t2_ref, out_ref):
    out_ref[...] = jnp.concatenate(
        [jnp.transpose(t1_ref[...]), jnp.transpose(t2_ref[...])], axis=-1)


def _make_repack(V, D):
    n_blocks = _PK // _BBK
    last_blk = pl.cdiv(V, _BBK) - 1

    return pl.pallas_call(
        _repack_body,
        grid=(n_blocks,),
        in_specs=[
            pl.BlockSpec((D, _BBK), lambda i: (0, i)),
            pl.BlockSpec((D, _BBK),
                         lambda i: (0, jnp.minimum(i + _PK // _BBK,
                                                   last_blk))),
        ],
        out_specs=pl.BlockSpec((_BBK, 2 * D), lambda i: (i, 0)),
        out_shape=jax.ShapeDtypeStruct((_PK, 2 * D), jnp.float32),
    )


def _make_sc_gather(D2, B):
    info = plsc.get_sparse_core_info()
    NC, NS = info.num_cores, info.num_subcores
    NW = NC * NS
    assert B % (NW * _IDX_CHUNK) == 0
    b_per_w = B // NW
    n_chunks = b_per_w // _IDX_CHUNK
    mesh = plsc.VectorSubcoreMesh(core_axis_name="c", subcore_axis_name="s")

    @functools.partial(
        pl.kernel,
        mesh=mesh,
        out_type=jax.ShapeDtypeStruct((B, D2), jnp.float32),
        scratch_types=[
            pltpu.VMEM((n_chunks, _IDX_CHUNK), jnp.int32),
            pltpu.VMEM((b_per_w, D2), jnp.float32),
            pltpu.SemaphoreType.DMA,
        ],
        compiler_params=pltpu.CompilerParams(use_tc_tiling_on_sc=True),
    )
    def gather(p_hbm, idx_hbm, out_hbm, idx_v, rows_v, sem):
        wid = lax.axis_index("s") * NC + lax.axis_index("c")
        base = wid * b_per_w
        pltpu.sync_copy(idx_hbm.at[pl.ds(wid * n_chunks, n_chunks)], idx_v)
        copies = []
        for j in range(n_chunks):
            copies.append(
                pltpu.make_async_copy(
                    p_hbm.at[idx_v.at[j]],
                    rows_v.at[pl.ds(j * _IDX_CHUNK, _IDX_CHUNK)],
                    sem,
                )
            )
        for cp in copies:
            cp.start()
        for cp in copies:
            cp.wait()
        pltpu.sync_copy(rows_v, out_hbm.at[pl.ds(base, b_per_w)])

    return gather


def _mlp_body(x_ref, hi_ref, gen_ref, w1a_ref, w1b_ref, b1_ref, w2_ref,
              b2_ref, out_ref):
    hi = jnp.transpose(hi_ref[...]) > 0
    emb = jnp.where(hi, x_ref[:, 64:128], x_ref[:, 0:64])
    h = jnp.dot(emb, w1a_ref[...], preferred_element_type=jnp.float32)
    h = h + jnp.dot(gen_ref[...], w1b_ref[...],
                    preferred_element_type=jnp.float32)
    h = jnp.maximum(h + b1_ref[...], 0.0)
    y = jnp.dot(h, w2_ref[...], preferred_element_type=jnp.float32)
    y = y + b2_ref[...]
    norm = jnp.sqrt(jnp.sum(y * y, axis=-1, keepdims=True))
    out_ref[...] = y / jnp.maximum(norm, 1e-12)


def kernel(item_ids, genre_vectors, table, W1, b1, W2, b2):
    B, = item_ids.shape
    V, D = table.shape
    G = genre_vectors.shape[1]
    H = W1.shape[1]

    packed = _make_repack(V, D)(table.T, table.T)

    ids = item_ids.astype(jnp.int32)
    p_idx = jnp.where(ids < _PK, ids, ids - _PK)
    idx2d = p_idx.reshape(B // _IDX_CHUNK, _IDX_CHUNK)
    x128 = _make_sc_gather(2 * D, B)(packed, idx2d)

    W1a = W1[:D]
    W1b = W1[D:]
    b1r = b1.reshape(1, H)
    b2r = b2.reshape(1, D)
    hi2d = (ids >= _PK).astype(jnp.int32).reshape(1, B)

    BB = 2048
    grid = (B // BB,)
    out = pl.pallas_call(
        _mlp_body,
        grid=grid,
        in_specs=[
            pl.BlockSpec((BB, 2 * D), lambda i: (i, 0)),
            pl.BlockSpec((1, BB), lambda i: (0, i)),
            pl.BlockSpec((BB, G), lambda i: (i, 0)),
            pl.BlockSpec((D, H), lambda i: (0, 0)),
            pl.BlockSpec((G, H), lambda i: (0, 0)),
            pl.BlockSpec((1, H), lambda i: (0, 0)),
            pl.BlockSpec((H, D), lambda i: (0, 0)),
            pl.BlockSpec((1, D), lambda i: (0, 0)),
        ],
        out_specs=pl.BlockSpec((BB, D), lambda i: (i, 0)),
        out_shape=jax.ShapeDtypeStruct((B, D), jnp.float32),
    )(x128, hi2d, genre_vectors, W1a, W1b, b1r, W2, b2r)
    return out


# speedup vs baseline: 2.4583x; 1.4277x over previous
import functools

import jax
import jax.numpy as jnp
from jax import lax
from jax.experimental import pallas as pl
from jax.experimental.pallas import tpu as pltpu
from jax.experimental.pallas import tpu_sc as plsc

_NSLOT = 4


def _make_sc_fetch(V, D, B):
    info = plsc.get_sparse_core_info()
    NC, NS = info.num_cores, info.num_subcores
    NW = NC * NS
    b_per_w = B // NW
    n_waves = b_per_w // _NSLOT
    t_max = (V - 1) // 128
    mesh = plsc.VectorSubcoreMesh(core_axis_name="c", subcore_axis_name="s")

    @functools.partial(
        pl.kernel,
        mesh=mesh,
        out_type=jax.ShapeDtypeStruct((B, D), jnp.float32),
        scratch_types=[
            pltpu.VMEM((b_per_w + 16,), jnp.int32),
            pltpu.VMEM((_NSLOT, D, 128), jnp.float32),
            pltpu.VMEM((b_per_w, D), jnp.float32),
            [pltpu.SemaphoreType.DMA] * _NSLOT,
        ],
        compiler_params=pltpu.CompilerParams(use_tc_tiling_on_sc=True,
                                             needs_layout_passes=False),
    )
    def fetch(tableT_hbm, idx_hbm, out_hbm, idx_v, blk_v, rows_v, sems):
        wid = lax.axis_index("s") * NC + lax.axis_index("c")
        base = wid * b_per_w
        pltpu.sync_copy(idx_hbm.at[pl.ds(base, b_per_w)],
                        idx_v.at[pl.ds(0, b_per_w)])
        rows16 = [jax.lax.broadcasted_iota(jnp.int32, (16,), 0) + 16 * k
                  for k in range(D // 16)]

        def issue(j, slot):
            r = idx_v[pl.ds(j, 16)][0]
            t = jnp.minimum(lax.shift_right_logical(r, 7), t_max)
            pltpu.make_async_copy(
                tableT_hbm.at[:, pl.ds(pl.multiple_of(t * 128, 128), 128)],
                blk_v.at[slot],
                sems[slot],
            ).start()

        def extract(j, slot):
            r = idx_v[pl.ds(j, 16)][0]
            col = jnp.broadcast_to(lax.bitwise_and(r, 127), (16,))
            jcol = jnp.broadcast_to(j, (16,))
            for k in range(D // 16):
                vals = plsc.load_gather(blk_v.at[slot], [rows16[k], col])
                plsc.store_scatter(rows_v, [jcol, rows16[k]], vals)

        for slot in range(_NSLOT):
            issue(slot, slot)

        def wave(w, carry):
            for slot in range(_NSLOT):
                j = w * _NSLOT + slot
                pltpu.make_async_copy(
                    tableT_hbm.at[:, pl.ds(0, 128)], blk_v.at[slot],
                    sems[slot],
                ).wait()
                extract(j, slot)

                @pl.when(w + 1 < n_waves)
                def _():
                    issue(j + _NSLOT, slot)
            return carry

        lax.fori_loop(0, n_waves, wave, 0)
        pltpu.sync_copy(rows_v, out_hbm.at[pl.ds(base, b_per_w)])

    return fetch


def _mlp_body(emb_ref, gen_ref, w1a_ref, w1b_ref, b1_ref, w2_ref, b2_ref,
              out_ref):
    h = jnp.dot(emb_ref[...], w1a_ref[...], preferred_element_type=jnp.float32)
    h = h + jnp.dot(gen_ref[...], w1b_ref[...],
                    preferred_element_type=jnp.float32)
    h = jnp.maximum(h + b1_ref[...], 0.0)
    y = jnp.dot(h, w2_ref[...], preferred_element_type=jnp.float32)
    y = y + b2_ref[...]
    norm = jnp.sqrt(jnp.sum(y * y, axis=-1, keepdims=True))
    out_ref[...] = y / jnp.maximum(norm, 1e-12)


def kernel(item_ids, genre_vectors, table, W1, b1, W2, b2):
    B, = item_ids.shape
    V, D = table.shape
    G = genre_vectors.shape[1]
    H = W1.shape[1]

    emb = _make_sc_fetch(V, D, B)(table.T, item_ids.astype(jnp.int32))

    W1a = W1[:D]
    W1b = W1[D:]
    b1r = b1.reshape(1, H)
    b2r = b2.reshape(1, D)

    BB = 2048
    grid = (B // BB,)
    out = pl.pallas_call(
        _mlp_body,
        grid=grid,
        in_specs=[
            pl.BlockSpec((BB, D), lambda i: (i, 0)),
            pl.BlockSpec((BB, G), lambda i: (i, 0)),
            pl.BlockSpec((D, H), lambda i: (0, 0)),
            pl.BlockSpec((G, H), lambda i: (0, 0)),
            pl.BlockSpec((1, H), lambda i: (0, 0)),
            pl.BlockSpec((H, D), lambda i: (0, 0)),
            pl.BlockSpec((1, D), lambda i: (0, 0)),
        ],
        out_specs=pl.BlockSpec((BB, D), lambda i: (i, 0)),
        out_shape=jax.ShapeDtypeStruct((B, D), jnp.float32),
    )(emb, genre_vectors, W1a, W1b, b1r, W2, b2r)
    return out


# speedup vs baseline: 2.8187x; 1.1466x over previous
import functools

import jax
import jax.numpy as jnp
from jax import lax
from jax.experimental import pallas as pl
from jax.experimental.pallas import tpu as pltpu
from jax.experimental.pallas import tpu_sc as plsc

_NSLOT = 8


def _make_sc_fetch(V, D, B):
    info = plsc.get_sparse_core_info()
    NC, NS = info.num_cores, info.num_subcores
    NW = NC * NS
    b_per_w = B // NW
    b_half = b_per_w // 2
    n_waves = b_half // _NSLOT
    t_max = (V - 1) // 128
    mesh = plsc.VectorSubcoreMesh(core_axis_name="c", subcore_axis_name="s")

    @functools.partial(
        pl.kernel,
        mesh=mesh,
        out_type=jax.ShapeDtypeStruct((B, D), jnp.float32),
        scratch_types=[
            pltpu.VMEM((b_per_w + 16,), jnp.int32),
            pltpu.VMEM((_NSLOT, D, 128), jnp.float32),
            pltpu.VMEM((b_half, D), jnp.float32),
            [pltpu.SemaphoreType.DMA] * _NSLOT,
        ],
        compiler_params=pltpu.CompilerParams(use_tc_tiling_on_sc=True,
                                             needs_layout_passes=False),
    )
    def fetch(tableT_hbm, idx_hbm, out_hbm, idx_v, blk_v, rows_v, sems):
        wid = lax.axis_index("s") * NC + lax.axis_index("c")
        base = wid * b_per_w
        pltpu.sync_copy(idx_hbm.at[pl.ds(base, b_per_w)],
                        idx_v.at[pl.ds(0, b_per_w)])
        rows16 = [jax.lax.broadcasted_iota(jnp.int32, (16,), 0) + 16 * k
                  for k in range(D // 16)]

        def issue(j, slot):
            r = idx_v[pl.ds(j, 16)][0]
            t = jnp.minimum(lax.shift_right_logical(r, 7), t_max)
            pltpu.make_async_copy(
                tableT_hbm.at[:, pl.ds(pl.multiple_of(t * 128, 128), 128)],
                blk_v.at[slot],
                sems[slot],
            ).start()

        def extract(j, j_rel, slot):
            r = idx_v[pl.ds(j, 16)][0]
            col = jnp.broadcast_to(lax.bitwise_and(r, 127), (16,))
            jcol = jnp.broadcast_to(j_rel, (16,))
            for k in range(D // 16):
                vals = plsc.load_gather(blk_v.at[slot], [rows16[k], col])
                plsc.store_scatter(rows_v, [jcol, rows16[k]], vals)

        for phase in range(2):
            off = phase * b_half

            for slot in range(_NSLOT):
                issue(off + slot, slot)

            def wave(w, carry):
                for slot in range(_NSLOT):
                    j_rel = w * _NSLOT + slot
                    j = off + j_rel
                    pltpu.make_async_copy(
                        tableT_hbm.at[:, pl.ds(0, 128)], blk_v.at[slot],
                        sems[slot],
                    ).wait()
                    extract(j, j_rel, slot)

                    @pl.when(w + 1 < n_waves)
                    def _():
                        issue(j + _NSLOT, slot)
                return carry

            lax.fori_loop(0, n_waves, wave, 0)
            pltpu.sync_copy(rows_v, out_hbm.at[pl.ds(base + off, b_half)])

    return fetch


def _mlp_body(emb_ref, gen_ref, w1a_ref, w1b_ref, b1_ref, w2_ref, b2_ref,
              out_ref):
    h = jnp.dot(emb_ref[...], w1a_ref[...], preferred_element_type=jnp.float32)
    h = h + jnp.dot(gen_ref[...], w1b_ref[...],
                    preferred_element_type=jnp.float32)
    h = jnp.maximum(h + b1_ref[...], 0.0)
    y = jnp.dot(h, w2_ref[...], preferred_element_type=jnp.float32)
    y = y + b2_ref[...]
    norm = jnp.sqrt(jnp.sum(y * y, axis=-1, keepdims=True))
    out_ref[...] = y / jnp.maximum(norm, 1e-12)


def kernel(item_ids, genre_vectors, table, W1, b1, W2, b2):
    B, = item_ids.shape
    V, D = table.shape
    G = genre_vectors.shape[1]
    H = W1.shape[1]

    emb = _make_sc_fetch(V, D, B)(table.T, item_ids.astype(jnp.int32))

    W1a = W1[:D]
    W1b = W1[D:]
    b1r = b1.reshape(1, H)
    b2r = b2.reshape(1, D)

    BB = 2048
    grid = (B // BB,)
    out = pl.pallas_call(
        _mlp_body,
        grid=grid,
        in_specs=[
            pl.BlockSpec((BB, D), lambda i: (i, 0)),
            pl.BlockSpec((BB, G), lambda i: (i, 0)),
            pl.BlockSpec((D, H), lambda i: (0, 0)),
            pl.BlockSpec((G, H), lambda i: (0, 0)),
            pl.BlockSpec((1, H), lambda i: (0, 0)),
            pl.BlockSpec((H, D), lambda i: (0, 0)),
            pl.BlockSpec((1, D), lambda i: (0, 0)),
        ],
        out_specs=pl.BlockSpec((BB, D), lambda i: (i, 0)),
        out_shape=jax.ShapeDtypeStruct((B, D), jnp.float32),
    )(emb, genre_vectors, W1a, W1b, b1r, W2, b2r)
    return out


# speedup vs baseline: 2.9695x; 1.0535x over previous
import functools

import jax
import jax.numpy as jnp
from jax import lax
from jax.experimental import pallas as pl
from jax.experimental.pallas import tpu as pltpu
from jax.experimental.pallas import tpu_sc as plsc

_NSLOT = 8


def _make_sc_fetch(V, D, B):
    info = plsc.get_sparse_core_info()
    NC, NS = info.num_cores, info.num_subcores
    NW = NC * NS
    b_per_w = B // NW
    b_half = b_per_w // 2
    n_waves = b_half // _NSLOT
    t_max = (V - 1) // 128
    mesh = plsc.VectorSubcoreMesh(core_axis_name="c", subcore_axis_name="s")

    @functools.partial(
        pl.kernel,
        mesh=mesh,
        out_type=jax.ShapeDtypeStruct((D, B), jnp.float32),
        scratch_types=[
            pltpu.VMEM((b_per_w + 16,), jnp.int32),
            pltpu.VMEM((_NSLOT, D, 128), jnp.float32),
            pltpu.VMEM((D, b_half), jnp.float32),
            [pltpu.SemaphoreType.DMA] * _NSLOT,
        ],
        compiler_params=pltpu.CompilerParams(use_tc_tiling_on_sc=True,
                                             needs_layout_passes=False),
    )
    def fetch(tableT_hbm, idx_hbm, out_hbm, idx_v, blk_v, rows_v, sems):
        wid = lax.axis_index("s") * NC + lax.axis_index("c")
        base = wid * b_per_w
        pltpu.sync_copy(idx_hbm.at[pl.ds(base, b_per_w)],
                        idx_v.at[pl.ds(0, b_per_w)])
        rows16 = [jax.lax.broadcasted_iota(jnp.int32, (16,), 0) + 16 * k
                  for k in range(D // 16)]

        def issue(j, slot):
            r = idx_v[pl.ds(j, 16)][0]
            t = jnp.minimum(lax.shift_right_logical(r, 7), t_max)
            pltpu.make_async_copy(
                tableT_hbm.at[:, pl.ds(pl.multiple_of(t * 128, 128), 128)],
                blk_v.at[slot],
                sems[slot],
            ).start()

        def extract(j, j_rel, slot):
            r = idx_v[pl.ds(j, 16)][0]
            col = jnp.broadcast_to(lax.bitwise_and(r, 127), (16,))
            jcol = jnp.broadcast_to(j_rel, (16,))
            for k in range(D // 16):
                vals = plsc.load_gather(blk_v.at[slot], [rows16[k], col])
                plsc.store_scatter(rows_v, [rows16[k], jcol], vals)

        for phase in range(2):
            off = phase * b_half

            for slot in range(_NSLOT):
                issue(off + slot, slot)

            def wave(w, carry):
                for slot in range(_NSLOT):
                    j_rel = w * _NSLOT + slot
                    j = off + j_rel
                    pltpu.make_async_copy(
                        tableT_hbm.at[:, pl.ds(0, 128)], blk_v.at[slot],
                        sems[slot],
                    ).wait()
                    extract(j, j_rel, slot)

                    @pl.when(w + 1 < n_waves)
                    def _():
                        issue(j + _NSLOT, slot)
                return carry

            lax.fori_loop(0, n_waves, wave, 0)
            pltpu.sync_copy(rows_v,
                            out_hbm.at[:, pl.ds(base + off, b_half)])

    return fetch


def _mlp_body(embT_ref, genT_ref, w1aT_ref, w1bT_ref, b1_ref, w2T_ref,
              b2_ref, outT_ref):
    hT = jnp.dot(w1aT_ref[...], embT_ref[...],
                 preferred_element_type=jnp.float32)
    hT = hT + jnp.dot(w1bT_ref[...], genT_ref[...],
                      preferred_element_type=jnp.float32)
    hT = jnp.maximum(hT + b1_ref[...], 0.0)
    yT = jnp.dot(w2T_ref[...], hT, preferred_element_type=jnp.float32)
    yT = yT + b2_ref[...]
    norm = jnp.sqrt(jnp.sum(yT * yT, axis=0, keepdims=True))
    outT_ref[...] = yT / jnp.maximum(norm, 1e-12)


def kernel(item_ids, genre_vectors, table, W1, b1, W2, b2):
    B, = item_ids.shape
    V, D = table.shape
    G = genre_vectors.shape[1]
    H = W1.shape[1]

    embT = _make_sc_fetch(V, D, B)(table.T, item_ids.astype(jnp.int32))

    genT = genre_vectors.T
    w1aT = W1[:D].T
    w1bT = W1[D:].T
    w2T = W2.T
    b1c = b1.reshape(H, 1)
    b2c = b2.reshape(D, 1)

    BB = 2048
    grid = (B // BB,)
    outT = pl.pallas_call(
        _mlp_body,
        grid=grid,
        in_specs=[
            pl.BlockSpec((D, BB), lambda i: (0, i)),
            pl.BlockSpec((G, BB), lambda i: (0, i)),
            pl.BlockSpec((H, D), lambda i: (0, 0)),
            pl.BlockSpec((H, G), lambda i: (0, 0)),
            pl.BlockSpec((H, 1), lambda i: (0, 0)),
            pl.BlockSpec((D, H), lambda i: (0, 0)),
            pl.BlockSpec((D, 1), lambda i: (0, 0)),
        ],
        out_specs=pl.BlockSpec((D, BB), lambda i: (0, i)),
        out_shape=jax.ShapeDtypeStruct((D, B), jnp.float32),
    )(embT, genT, w1aT, w1bT, b1c, w2T, b2c)
    return outT.T
